# baseline (speedup 1.0000x reference)
import functools

import jax
import jax.numpy as jnp
from jax import lax
from jax.experimental import pallas as pl
from jax.experimental.pallas import tpu as pltpu
from jax.experimental.pallas import tpu_sc as plsc

N_EVENTS = 16384
TOTAL = N_EVENTS * 48

NC, NS, L = 2, 16, 16
NW = NC * NS
EV_PER_W = N_EVENTS // NW
W_WORDS = EV_PER_W * 48
CHUNK = 4096
PLANE = N_EVENTS * 8

SEGS = [(0, 6), (6, 9), (9, 12), (12, 15)]


HALF = CHUNK // 2


def _sc_body(x_hbm, out_hbm, in_v, out_v, sem_a, sem_b, sem_o):
    wid = lax.axis_index("s") * NC + lax.axis_index("c")
    wbase = wid * CHUNK

    def in_copies(h, sem):
        return [pltpu.async_copy(
            x_hbm.at[pl.ds(p * PLANE + wbase + h * HALF, HALF)],
            in_v.at[pl.ds(p * CHUNK + h * HALF, HALF)], sem)
            for p in range(6)]

    def out_copies(h):
        return [pltpu.async_copy(
            out_v.at[pl.ds(p * CHUNK + h * HALF, HALF)],
            out_hbm.at[pl.ds(p * PLANE + wbase + h * HALF, HALF)], sem_o)
            for p in range(6)]

    cp_a = in_copies(0, sem_a)
    cp_b = in_copies(1, sem_b)

    i16 = lax.iota(jnp.int32, L)
    one = jnp.ones((L,), jnp.int32)
    zf = jnp.zeros((L,), jnp.float32)
    zi = jnp.zeros((L,), jnp.int32)
    lut = ((i16 >> 3) << 12) + ((i16 & 7) << 7)

    def block(m):
        mbase = (m >> 3) * 1024 + (m & 7) * 16

        def off(i, c):
            return mbase + (c * 2 + (i >> 3)) * CHUNK + (i & 7) * 128

        v = [[in_v[pl.ds(off(i, c), L)] for c in range(3)] for i in range(16)]
        msk = [v[i][0] > 0.0 for i in range(15)]
        mi = [jnp.where(msk[i], one, zi) for i in range(15)]

        rank = [None] * 15
        for s, e in SEGS:
            acc = zi
            for i in range(s, e):
                rank[i] = acc
                acc = acc + mi[i]

        mvec = mbase + i16
        for s, e in SEGS:
            for i in range(s, e):
                dst = rank[i] + jnp.where(msk[i], s, e + s - 1 - i)
                g = jnp.take(lut, dst) + mvec
                for c in range(3):
                    val = jnp.where(msk[i], v[i][c], zf)
                    plsc.store_scatter(out_v, [g + c * (2 * CHUNK)], val)
        for c in range(3):
            out_v[pl.ds(off(15, c), L)] = v[15][c]

    def run_half(lo, hi):
        def body(it, car):
            block(it * 2)
            block(it * 2 + 1)
            return car
        lax.fori_loop(lo // 2, hi // 2, body, jnp.int32(0))

    for cp in cp_a:
        cp.wait()
    run_half(0, 16)
    out0 = out_copies(0)
    for cp in cp_b:
        cp.wait()
    run_half(16, 32)
    out1 = out_copies(1)
    for cp in out0 + out1:
        cp.wait()


@jax.jit
def _reorder(xf):
    mesh = plsc.VectorSubcoreMesh(core_axis_name="c", subcore_axis_name="s",
                                  num_cores=NC, num_subcores=NS)
    return pl.kernel(
        _sc_body,
        out_type=jax.ShapeDtypeStruct((TOTAL,), jnp.float32),
        mesh=mesh,
        scratch_types=[
            pltpu.VMEM((W_WORDS,), jnp.float32),
            pltpu.VMEM((W_WORDS,), jnp.float32),
            pltpu.SemaphoreType.DMA,
            pltpu.SemaphoreType.DMA,
            pltpu.SemaphoreType.DMA,
        ],
        compiler_params=pltpu.CompilerParams(needs_layout_passes=False),
    )(xf)


def kernel(inputs):
    xf = (inputs.transpose(2, 1, 0).reshape(3, 2, 8, 128, 128)
          .transpose(0, 1, 3, 2, 4).reshape(TOTAL))
    of = _reorder(xf)
    return (of.reshape(3, 2, 128, 8, 128).transpose(0, 1, 3, 2, 4)
            .reshape(3, 16, N_EVENTS).transpose(2, 1, 0))


# speedup vs baseline: 1.0024x; 1.0024x over previous
import jax
import jax.numpy as jnp
from jax import lax
from jax.experimental import pallas as pl
from jax.experimental.pallas import tpu as pltpu
from jax.experimental.pallas import tpu_sc as plsc

N_EVENTS = 16384
TOTAL = N_EVENTS * 48

NC, NS, L = 2, 16, 16
NW = NC * NS
EV_PER_W = N_EVENTS // NW
W_WORDS = EV_PER_W * 48
CHUNK = 4096
PLANE = N_EVENTS * 8

SEGS = [(0, 6), (6, 9), (9, 12), (12, 15)]


HALF = CHUNK // 2


def _sc_body(x_hbm, out_hbm, in_v, out_v, sem_a, sem_b, sem_o):
    wid = lax.axis_index("s") * NC + lax.axis_index("c")
    wbase = wid * CHUNK

    def in_copies(h, sem):
        return [pltpu.async_copy(
            x_hbm.at[pl.ds(p * PLANE + wbase + h * HALF, HALF)],
            in_v.at[pl.ds(p * CHUNK + h * HALF, HALF)], sem)
            for p in range(6)]

    def out_copies(h):
        return [pltpu.async_copy(
            out_v.at[pl.ds(p * CHUNK + h * HALF, HALF)],
            out_hbm.at[pl.ds(p * PLANE + wbase + h * HALF, HALF)], sem_o)
            for p in range(6)]

    cp_a = in_copies(0, sem_a)
    cp_b = in_copies(1, sem_b)

    i16 = lax.iota(jnp.int32, L)
    one = jnp.ones((L,), jnp.int32)
    zf = jnp.zeros((L,), jnp.float32)
    zi = jnp.zeros((L,), jnp.int32)
    lut = ((i16 >> 3) << 12) + ((i16 & 7) << 7)

    def block(m):
        mbase = (m >> 3) * 1024 + (m & 7) * 16

        def off(i, c):
            return mbase + (c * 2 + (i >> 3)) * CHUNK + (i & 7) * 128

        v = [[in_v[pl.ds(off(i, c), L)] for c in range(3)] for i in range(16)]
        msk = [v[i][0] > 0.0 for i in range(15)]
        mi = [jnp.where(msk[i], one, zi) for i in range(15)]

        rank = [None] * 15
        for s, e in SEGS:
            acc = zi
            for i in range(s, e):
                rank[i] = acc
                acc = acc + mi[i]

        mvec = mbase + i16
        for s, e in SEGS:
            for i in range(s, e):
                dst = rank[i] + jnp.where(msk[i], s, e + s - 1 - i)
                g = jnp.take(lut, dst) + mvec
                for c in range(3):
                    val = jnp.where(msk[i], v[i][c], zf)
                    plsc.store_scatter(out_v, [g + c * (2 * CHUNK)], val)
        for c in range(3):
            out_v[pl.ds(off(15, c), L)] = v[15][c]

    def run_half(lo, hi):
        def body(it, car):
            block(it * 2)
            block(it * 2 + 1)
            return car
        lax.fori_loop(lo // 2, hi // 2, body, jnp.int32(0))

    for cp in cp_a:
        cp.wait()
    run_half(0, 16)
    out0 = out_copies(0)
    for cp in cp_b:
        cp.wait()
    run_half(16, 32)
    out1 = out_copies(1)
    for cp in out0 + out1:
        cp.wait()


@jax.jit
def _reorder(xf):
    mesh = plsc.VectorSubcoreMesh(core_axis_name="c", subcore_axis_name="s",
                                  num_cores=NC, num_subcores=NS)
    return pl.kernel(
        _sc_body,
        out_type=jax.ShapeDtypeStruct((TOTAL,), jnp.float32),
        mesh=mesh,
        scratch_types=[
            pltpu.VMEM((W_WORDS,), jnp.float32),
            pltpu.VMEM((W_WORDS,), jnp.float32),
            pltpu.SemaphoreType.DMA,
            pltpu.SemaphoreType.DMA,
            pltpu.SemaphoreType.DMA,
        ],
        compiler_params=pltpu.CompilerParams(needs_layout_passes=False),
    )(xf)


def kernel(inputs):
    xf = (inputs.transpose(2, 1, 0).reshape(3, 2, 8, 128, 128)
          .transpose(0, 1, 3, 2, 4).reshape(TOTAL))
    of = _reorder(xf)
    return (of.reshape(3, 2, 128, 8, 128).transpose(0, 1, 3, 2, 4)
            .reshape(3, 16, N_EVENTS).transpose(2, 1, 0))
